# baseline (speedup 1.0000x reference)
import jax
import jax.numpy as jnp
from jax.experimental import pallas as pl

_BM = 512


def _fused_block(x_ref, a_ref, o_ref):
    t = jnp.dot(a_ref[...], x_ref[...], preferred_element_type=jnp.float32)
    o_ref[...] = (t > 0.5).astype(jnp.float32)


def kernel(x, a):
    m, k = a.shape
    n = x.shape[1]
    return pl.pallas_call(
        _fused_block,
        grid=(m // _BM,),
        in_specs=[
            pl.BlockSpec((k, n), lambda i: (0, 0)),
            pl.BlockSpec((_BM, k), lambda i: (i, 0)),
        ],
        out_specs=pl.BlockSpec((_BM, n), lambda i: (i, 0)),
        out_shape=jax.ShapeDtypeStruct((m, n), jnp.float32),
    )(x, a)


# speedup vs baseline: 1.0047x; 1.0047x over previous
import jax
import jax.numpy as jnp
from jax.experimental import pallas as pl
from jax.experimental.pallas import tpu as pltpu

_BM = 512


def _fused_block(x_ref, a_ref, o_ref):
    t = jnp.dot(a_ref[...], x_ref[...], preferred_element_type=jnp.float32)
    o_ref[...] = (t > 0.5).astype(jnp.float32)


def kernel(x, a):
    m, k = a.shape
    n = x.shape[1]
    return pl.pallas_call(
        _fused_block,
        grid=(m // _BM,),
        in_specs=[
            pl.BlockSpec((k, n), lambda i: (0, 0)),
            pl.BlockSpec((_BM, k), lambda i: (i, 0)),
        ],
        out_specs=pl.BlockSpec((_BM, n), lambda i: (i, 0)),
        out_shape=jax.ShapeDtypeStruct((m, n), jnp.float32),
        compiler_params=pltpu.CompilerParams(
            dimension_semantics=("parallel",),
        ),
    )(x, a)


# speedup vs baseline: 1.0244x; 1.0196x over previous
import jax
import jax.numpy as jnp
from jax.experimental import pallas as pl
from jax.experimental.pallas import tpu as pltpu

_BM = 256


def _fused_block(x_ref, a_ref, o_ref):
    t = jnp.dot(a_ref[...], x_ref[...], preferred_element_type=jnp.float32)
    o_ref[...] = (t > 0.5).astype(jnp.float32)


def kernel(x, a):
    m, k = a.shape
    n = x.shape[1]
    return pl.pallas_call(
        _fused_block,
        grid=(m // _BM,),
        in_specs=[
            pl.BlockSpec((k, n), lambda i: (0, 0)),
            pl.BlockSpec((_BM, k), lambda i: (i, 0)),
        ],
        out_specs=pl.BlockSpec((_BM, n), lambda i: (i, 0)),
        out_shape=jax.ShapeDtypeStruct((m, n), jnp.float32),
        compiler_params=pltpu.CompilerParams(
            dimension_semantics=("parallel",),
        ),
    )(x, a)
